# baseline (speedup 1.0000x reference)
import functools

import numpy as np
import jax
import jax.numpy as jnp
from jax import lax
from jax.experimental import pallas as pl
from jax.experimental.pallas import tpu as pltpu
from jax.experimental.pallas import tpu_sc as plsc

_NUM_JOKERS = 150
_EPS = 1e-5


def _table_body(emb_ref, pos_ref, gam_ref, bet_ref, f_ref):
    ni = emb_ref.shape[0]
    mm = pos_ref.shape[0]
    d = emb_ref.shape[1]
    emb = emb_ref[...]
    row = lax.broadcasted_iota(jnp.int32, (ni, d), 0)
    emb = jnp.where(row == 0, 0.0, emb)
    pos = pos_ref[...]
    x = jnp.concatenate([emb + pos[p:p + 1] for p in range(mm)], axis=0)
    mu = jnp.mean(x, axis=-1, keepdims=True)
    xc = x - mu
    var = jnp.mean(xc * xc, axis=-1, keepdims=True)
    y = xc * lax.rsqrt(var + _EPS)
    f_ref[...] = y * gam_ref[...] + bet_ref[...]


def _table(emb_table, pos_table, ln_gamma, ln_beta, interpret=False):
    ni, d = emb_table.shape
    mm = pos_table.shape[0]
    return pl.pallas_call(
        _table_body,
        out_shape=jax.ShapeDtypeStruct((mm * ni, d), jnp.float32),
        interpret=interpret,
    )(emb_table, pos_table, ln_gamma.reshape(1, d), ln_beta.reshape(1, d))


def _idxmask_body(boss_ref, act_ref, jok_ref, emp_ref, idx_ref, msk_ref):
    ni = 179
    mm = msk_ref.shape[1]
    r_i = lax.broadcasted_iota(jnp.int32, (128, 128), 0)
    c_i = lax.broadcasted_iota(jnp.int32, (128, 128), 1)
    eye = (r_i == c_i).astype(jnp.float32)

    def tr_in(x):
        return lax.dot_general(x.astype(jnp.float32), eye,
                               (((0,), (0,)), ((), ())),
                               preferred_element_type=jnp.float32)

    act = act_ref[...][0] != 0
    acti = act.astype(jnp.int32)
    boss = boss_ref[...][0] + _NUM_JOKERS
    jt = tr_in(jok_ref[...]).astype(jnp.int32)
    i0 = jnp.where(act, boss, jt[0:1])
    imid = jnp.where(act, jt[0:mm - 2], jt[1:mm - 1])
    i10 = jnp.where(act, jt[mm - 2:mm - 1], 0)
    idx = jnp.concatenate([i0, imid, i10], axis=0)
    poff = lax.broadcasted_iota(jnp.int32, idx.shape, 0) * ni
    idx_ref[...] = (idx + poff)[:, None, None, :]

    real = 1 - tr_in(emp_ref[...]).astype(jnp.int32)
    m0 = acti | real[0:1]
    mmid = jnp.where(act, real[0:mm - 2], real[1:mm - 1])
    m10 = acti & real[mm - 2:mm - 1]
    rest = jnp.concatenate([mmid, m10], axis=0)
    rest_any = jnp.max(rest, axis=0, keepdims=True)
    m0 = m0 | (1 - rest_any)
    m = jnp.concatenate([m0, mmid, m10], axis=0)
    mt = lax.dot_general(eye, m.astype(jnp.float32),
                         (((1,), (1,)), ((), ())),
                         preferred_element_type=jnp.float32)
    msk_ref[...] = mt.astype(jnp.int32)


def _idxmask(boss2, act2, jok, emp, interpret=False):
    bb, nj = jok.shape
    mm = nj + 1
    nblk = bb // 128
    return pl.pallas_call(
        _idxmask_body,
        grid=(nblk,),
        in_specs=[
            pl.BlockSpec((1, 1, 128), lambda i: (i, 0, 0)),
            pl.BlockSpec((1, 1, 128), lambda i: (i, 0, 0)),
            pl.BlockSpec((128, nj), lambda i: (i, 0)),
            pl.BlockSpec((128, nj), lambda i: (i, 0)),
        ],
        out_specs=[
            pl.BlockSpec((mm, 1, 1, 128), lambda i: (0, i, 0, 0)),
            pl.BlockSpec((128, mm), lambda i: (i, 0)),
        ],
        out_shape=[
            jax.ShapeDtypeStruct((mm, nblk, 1, 128), jnp.int32),
            jax.ShapeDtypeStruct((bb, mm), jnp.int32),
        ],
        interpret=interpret,
    )(boss2.reshape(bb // 128, 1, 128), act2.reshape(bb // 128, 1, 128),
      jok, emp)


_NBUF = 2
_CHUNK = 128


def _sc_gather_scatter(table, idx1, dst2):
    n_rows = idx1.shape[0]
    d = table.shape[1]
    info = plsc.get_sparse_core_info()
    nw = info.num_cores * info.num_subcores
    rows_per_w = n_rows // nw
    slabs_per_w = rows_per_w // _CHUNK
    stage = ((slabs_per_w + 7) // 8) * 8 + 8
    stage = min(stage, dst2.shape[0])
    mesh = plsc.VectorSubcoreMesh(core_axis_name="c", subcore_axis_name="s")

    @functools.partial(
        pl.kernel,
        out_type=jax.ShapeDtypeStruct((n_rows, d), jnp.float32),
        mesh=mesh,
        scratch_types=[
            pltpu.VMEM((rows_per_w,), jnp.int32),
            pltpu.VMEM((stage, _CHUNK), jnp.int32),
        ] + [pltpu.VMEM((_CHUNK, d), jnp.float32)] * _NBUF
          + [pltpu.SemaphoreType.DMA] * (2 * _NBUF),
    )
    def k(table_hbm, idx_hbm, dst_hbm, out_hbm, idx_v, dst_v, *rest):
        bufs = rest[:_NBUF]
        gsems = rest[_NBUF:2 * _NBUF]
        osems = rest[2 * _NBUF:]
        wid = lax.axis_index("s") * info.num_cores + lax.axis_index("c")
        base = wid * slabs_per_w
        a0 = pl.multiple_of((base // 8) * 8, 8)
        a0 = jnp.minimum(a0, dst2.shape[0] - stage)
        a0 = pl.multiple_of(a0, 8)
        off = base - a0
        pltpu.sync_copy(
            idx_hbm.at[pl.ds(pl.multiple_of(wid * rows_per_w, 8),
                             rows_per_w)], idx_v)
        pltpu.sync_copy(dst_hbm.at[pl.ds(a0, stage)], dst_v)

        def gather(s, b):
            pltpu.async_copy(
                table_hbm.at[idx_v.at[pl.ds(s * _CHUNK, _CHUNK)]],
                bufs[b], gsems[b])

        def gwait(s, b):
            pltpu.make_async_copy(
                table_hbm.at[idx_v.at[pl.ds(s * _CHUNK, _CHUNK)]],
                bufs[b], gsems[b]).wait()

        for b in range(_NBUF):
            gather(b, b)

        def wave(w, carry):
            for b in range(_NBUF):
                s = w * _NBUF + b
                gwait(s, b)
                pltpu.async_copy(bufs[b], out_hbm.at[dst_v.at[off + s]],
                                 osems[b])
            for b in range(_NBUF):
                s = w * _NBUF + b
                pltpu.make_async_copy(bufs[b], out_hbm.at[dst_v.at[off + s]],
                                      osems[b]).wait()
                ns = s + _NBUF

                @pl.when(ns < slabs_per_w)
                def _():
                    gather(ns, b)
            return carry

        lax.fori_loop(0, slabs_per_w // _NBUF, wave, 0)

    return k(table, idx1, dst2)


def kernel(boss_id, boss_is_active, joker_ids, joker_is_empty,
           emb_table, pos_table, ln_gamma, ln_beta):
    b = boss_id.shape[0]
    ni, d = emb_table.shape
    mm = joker_ids.shape[1] + 1
    r = b // 128

    boss2 = boss_id.astype(jnp.int32).reshape(r, 128)
    act2 = boss_is_active.astype(jnp.int32).reshape(r, 128)
    jok = joker_ids.astype(jnp.int32)
    emp = joker_is_empty.astype(jnp.int32)

    table = _table(emb_table, pos_table, ln_gamma, ln_beta)
    idx_t, msk = _idxmask(boss2, act2, jok, emp)

    idx_flat = idx_t.reshape(mm * b)
    mask = msk.astype(bool)

    q = np.arange(mm * b, dtype=np.int64)
    dst = ((q % b) * mm + q // b).astype(np.int32).reshape(mm * b // _CHUNK,
                                                           _CHUNK)
    out = _sc_gather_scatter(table, idx_flat, jnp.asarray(dst))
    return out.reshape(b, mm, d), mask


# speedup vs baseline: 1.1567x; 1.1567x over previous
import functools

import numpy as np
import jax
import jax.numpy as jnp
from jax import lax
from jax.experimental import pallas as pl
from jax.experimental.pallas import tpu as pltpu
from jax.experimental.pallas import tpu_sc as plsc

_NUM_JOKERS = 150
_EPS = 1e-5


def _prep_body(emb_ref, pos_ref, gam_ref, bet_ref, boss_ref, act_ref,
               jid_ref, f_ref, idx_ref):
    ni = emb_ref.shape[0]
    mm = pos_ref.shape[0]
    d = emb_ref.shape[1]
    emb = emb_ref[...]
    row = lax.broadcasted_iota(jnp.int32, (ni, d), 0)
    emb = jnp.where(row == 0, 0.0, emb)
    x = emb[None, :, :] + pos_ref[...][:, None, :]
    mu = jnp.mean(x, axis=-1, keepdims=True)
    xc = x - mu
    var = jnp.mean(xc * xc, axis=-1, keepdims=True)
    y = xc * lax.rsqrt(var + _EPS)
    f_ref[...] = y * gam_ref[...][None, :, :] + bet_ref[...][None, :, :]

    act = act_ref[...] != 0
    boss = boss_ref[...] + _NUM_JOKERS
    j = jid_ref[...]
    i0 = jnp.where(act, boss, j[0])[None]
    imid = jnp.where(act[None], j[0:mm - 2], j[1:mm - 1])
    i10 = jnp.where(act, j[mm - 2], 0)[None]
    idx = jnp.concatenate([i0, imid, i10], axis=0)
    poff = lax.broadcasted_iota(jnp.int32, idx.shape, 0) * ni
    idx_ref[...] = idx + poff


def _prep(emb_table, pos_table, ln_gamma, ln_beta, boss2, act2, jt,
          interpret=False):
    ni, d = emb_table.shape
    mm = pos_table.shape[0]
    r = boss2.shape[0]
    return pl.pallas_call(
        _prep_body,
        out_shape=[
            jax.ShapeDtypeStruct((mm, ni, d), jnp.float32),
            jax.ShapeDtypeStruct((mm, r, 128), jnp.int32),
        ],
        interpret=interpret,
    )(emb_table, pos_table, ln_gamma.reshape(1, d), ln_beta.reshape(1, d),
      boss2, act2, jt)


_MBLK = 1024


def _mask_body(actn_ref, empn_ref, msk_ref):
    mm = msk_ref.shape[1]
    actn = actn_ref[...]
    actb = actn != 0
    real = (empn_ref[...] == 0).astype(jnp.int32)
    m0 = actn | real[:, 0:1]
    mmid = jnp.where(actb, real[:, 0:mm - 2], real[:, 1:mm - 1])
    m10 = actn & real[:, mm - 2:mm - 1]
    rest = jnp.concatenate([mmid, m10], axis=1)
    rest_any = jnp.max(rest, axis=1, keepdims=True)
    m0 = m0 | (1 - rest_any)
    msk_ref[...] = jnp.concatenate([m0, mmid, m10], axis=1)


def _mask(actn, empn, interpret=False):
    bb, nj = empn.shape
    mm = nj + 1
    return pl.pallas_call(
        _mask_body,
        grid=(bb // _MBLK,),
        in_specs=[
            pl.BlockSpec((_MBLK, 1), lambda i: (i, 0)),
            pl.BlockSpec((_MBLK, nj), lambda i: (i, 0)),
        ],
        out_specs=pl.BlockSpec((_MBLK, mm), lambda i: (i, 0)),
        out_shape=jax.ShapeDtypeStruct((bb, mm), jnp.int32),
        interpret=interpret,
    )(actn, empn)


_NBUF = 2
_CHUNK = 128


def _sc_gather_scatter(table, idx1, dst2):
    n_rows = idx1.shape[0]
    d = table.shape[1]
    info = plsc.get_sparse_core_info()
    nw = info.num_cores * info.num_subcores
    rows_per_w = n_rows // nw
    slabs_per_w = rows_per_w // _CHUNK
    stage = ((slabs_per_w + 7) // 8) * 8 + 8
    stage = min(stage, dst2.shape[0])
    mesh = plsc.VectorSubcoreMesh(core_axis_name="c", subcore_axis_name="s")

    @functools.partial(
        pl.kernel,
        out_type=jax.ShapeDtypeStruct((n_rows, d), jnp.float32),
        mesh=mesh,
        scratch_types=[
            pltpu.VMEM((rows_per_w,), jnp.int32),
            pltpu.VMEM((stage, _CHUNK), jnp.int32),
        ] + [pltpu.VMEM((_CHUNK, d), jnp.float32)] * _NBUF
          + [pltpu.SemaphoreType.DMA] * (2 * _NBUF),
    )
    def k(table_hbm, idx_hbm, dst_hbm, out_hbm, idx_v, dst_v, *rest):
        bufs = rest[:_NBUF]
        gsems = rest[_NBUF:2 * _NBUF]
        osems = rest[2 * _NBUF:]
        wid = lax.axis_index("s") * info.num_cores + lax.axis_index("c")
        base = wid * slabs_per_w
        a0 = pl.multiple_of((base // 8) * 8, 8)
        a0 = jnp.minimum(a0, dst2.shape[0] - stage)
        a0 = pl.multiple_of(a0, 8)
        off = base - a0
        pltpu.sync_copy(
            idx_hbm.at[pl.ds(pl.multiple_of(wid * rows_per_w, 8),
                             rows_per_w)], idx_v)
        pltpu.sync_copy(dst_hbm.at[pl.ds(a0, stage)], dst_v)

        def gather(s, b):
            pltpu.async_copy(
                table_hbm.at[idx_v.at[pl.ds(s * _CHUNK, _CHUNK)]],
                bufs[b], gsems[b])

        def gwait(s, b):
            pltpu.make_async_copy(
                table_hbm.at[idx_v.at[pl.ds(s * _CHUNK, _CHUNK)]],
                bufs[b], gsems[b]).wait()

        for b in range(_NBUF):
            gather(b, b)

        def wave(w, carry):
            for b in range(_NBUF):
                s = w * _NBUF + b
                gwait(s, b)
                pltpu.async_copy(bufs[b], out_hbm.at[dst_v.at[off + s]],
                                 osems[b])
            for b in range(_NBUF):
                s = w * _NBUF + b
                pltpu.make_async_copy(bufs[b], out_hbm.at[dst_v.at[off + s]],
                                      osems[b]).wait()
                ns = s + _NBUF

                @pl.when(ns < slabs_per_w)
                def _():
                    gather(ns, b)
            return carry

        lax.fori_loop(0, slabs_per_w // _NBUF, wave, 0)

    return k(table, idx1, dst2)


def kernel(boss_id, boss_is_active, joker_ids, joker_is_empty,
           emb_table, pos_table, ln_gamma, ln_beta):
    b = boss_id.shape[0]
    ni, d = emb_table.shape
    mm = joker_ids.shape[1] + 1
    r = b // 128

    boss2 = boss_id.astype(jnp.int32).reshape(r, 128)
    act2 = boss_is_active.astype(jnp.int32).reshape(r, 128)
    jt = joker_ids.astype(jnp.int32).T.reshape(mm - 1, r, 128)
    actn = boss_is_active.astype(jnp.int32).reshape(b, 1)
    empn = joker_is_empty.astype(jnp.int32)

    f, idx_t = _prep(emb_table, pos_table, ln_gamma, ln_beta,
                     boss2, act2, jt)
    msk = _mask(actn, empn)

    table = f.reshape(mm * ni, d)
    idx_flat = idx_t.reshape(mm * b)
    mask = msk.astype(bool)

    q = np.arange(mm * b, dtype=np.int64)
    dst = ((q % b) * mm + q // b).astype(np.int32).reshape(mm * b // _CHUNK,
                                                           _CHUNK)
    out = _sc_gather_scatter(table, idx_flat, jnp.asarray(dst))
    return out.reshape(b, mm, d), mask
